# baseline (speedup 1.0000x reference)
import functools

import jax
import jax.numpy as jnp
from jax import lax
from jax.experimental import pallas as pl
from jax.experimental.pallas import tpu as pltpu
from jax.experimental.pallas import tpu_sc as plsc

ROWS = 128
COLS = 32768
LANES = 16
N_WORKERS = 32
ROWS_PER_WORKER = ROWS // N_WORKERS
VREGS_PER_ROW = COLS // LANES
UNROLL = 16
K_ACC = 8
CHUNK = 8192
CHUNK_V = CHUNK // LANES
CPR = COLS // CHUNK
NBUF = 3
N_ITERS_CHUNK = CHUNK // (UNROLL * LANES)

_NEG_INF = float("-inf")


def _shuffle(v, idx):
    return v.at[idx].get(mode="promise_in_bounds")


def _butterfly(v, iota, op):
    for k in (1, 2, 4, 8):
        v = op(v, _shuffle(v, iota ^ k))
    return v


def _sc_body(x_hbm, out_hbm, buf, res_vmem, sem0, sem1, sem2):
    cid = lax.axis_index("c")
    sid = lax.axis_index("s")
    wid = cid * 16 + sid

    iota = lax.iota(jnp.int32, LANES)
    res = jnp.zeros((LANES,), jnp.float32)

    sems = (sem0, sem1, sem2)
    n_chunks = ROWS_PER_WORKER * CPR
    copies = [None] * NBUF
    row0 = wid * ROWS_PER_WORKER

    def _issue(g):
        slot = g % NBUF
        return pltpu.async_copy(
            x_hbm.at[row0 + g // CPR, pl.ds((g % CPR) * CHUNK_V, CHUNK_V)],
            buf.at[pl.ds(slot * CHUNK_V, CHUNK_V)],
            sems[slot],
        )

    for p in range(NBUF - 1):
        copies[p] = _issue(p)

    ninf = jnp.full((LANES,), _NEG_INF)
    zero = jnp.zeros((LANES,), jnp.float32)

    for j in range(ROWS_PER_WORKER):
        carry = (ninf,) * (2 * K_ACC) + (zero,) * K_ACC
        for c in range(CPR):
            g = j * CPR + c
            if g + NBUF - 1 < n_chunks:
                copies[(g + NBUF - 1) % NBUF] = _issue(g + NBUF - 1)
            copies[g % NBUF].wait()
            slot_base = (g % NBUF) * CHUNK_V

            def fused(i, carry):
                m1s = list(carry[:K_ACC])
                m2s = list(carry[K_ACC:2 * K_ACC])
                accs = list(carry[2 * K_ACC:])
                base = slot_base + i * UNROLL
                for t in range(UNROLL):
                    k = t % K_ACC
                    v = buf[base + t]
                    m2s[k] = jnp.maximum(m2s[k], jnp.minimum(m1s[k], v))
                    m1s[k] = jnp.maximum(m1s[k], v)
                    accs[k] = accs[k] + jnp.exp(v)
                return tuple(m1s) + tuple(m2s) + tuple(accs)

            carry = lax.fori_loop(0, N_ITERS_CHUNK, fused, carry)

        pairs = [(carry[k], carry[K_ACC + k]) for k in range(K_ACC)]
        while len(pairs) > 1:
            nxt_pairs = []
            for p in range(0, len(pairs), 2):
                (a1, a2), (b1, b2) = pairs[p], pairs[p + 1]
                nxt_pairs.append((
                    jnp.maximum(a1, b1),
                    jnp.maximum(jnp.minimum(a1, b1), jnp.maximum(a2, b2)),
                ))
            pairs = nxt_pairs
        m1v, m2v = pairs[0]

        m1b = _butterfly(m1v, iota, jnp.maximum)
        first = _butterfly(
            jnp.where(m1v == m1b, iota, jnp.int32(LANES)), iota, jnp.minimum
        )
        m2b = _butterfly(jnp.where(iota == first, m2v, m1v), iota, jnp.maximum)

        accs = list(carry[2 * K_ACC:])
        while len(accs) > 1:
            accs = [accs[p] + accs[p + 1] for p in range(0, len(accs), 2)]
        sv = _butterfly(accs[0], iota, jnp.add)

        rv = (jnp.exp(m1b + m2b) * jnp.float32(4.0)) / (sv * sv)
        res = jnp.where(iota == j, rv, res)

    res_vmem[...] = res
    pltpu.sync_copy(res_vmem, out_hbm.at[wid])


def _sc_call(x):
    mesh = plsc.VectorSubcoreMesh(core_axis_name="c", subcore_axis_name="s")
    fn = functools.partial(
        pl.kernel,
        mesh=mesh,
        out_type=jax.ShapeDtypeStruct((N_WORKERS, LANES), jnp.float32),
        scratch_types=[
            pltpu.VMEM((NBUF * CHUNK_V, LANES), jnp.float32),
            pltpu.VMEM((LANES,), jnp.float32),
            pltpu.SemaphoreType.DMA,
            pltpu.SemaphoreType.DMA,
            pltpu.SemaphoreType.DMA,
        ],
        compiler_params=pltpu.CompilerParams(use_tc_tiling_on_sc=False),
    )(_sc_body)
    return fn(x)


def kernel(inputs):
    x3 = inputs.reshape(ROWS, COLS // LANES, LANES)
    out32 = _sc_call(x3)
    return out32[:, :ROWS_PER_WORKER].reshape(ROWS, 1)


# speedup vs baseline: 2.9451x; 2.9451x over previous
import functools

import jax
import jax.numpy as jnp
from jax import lax
from jax.experimental import pallas as pl
from jax.experimental.pallas import tpu as pltpu
from jax.experimental.pallas import tpu_sc as plsc

ROWS = 128
COLS = 32768
LANES = 16
N_WORKERS = 32
ROWS_PER_WORKER = ROWS // N_WORKERS
VREGS_PER_ROW = COLS // LANES
K_ACC = 8
CHUNK = 8192
BLK = 1024
CHUNK_B = CHUNK // BLK
CPR = COLS // CHUNK
NBUF = 3
UNROLL = BLK // LANES
N_ITERS_CHUNK = CHUNK // BLK

_NEG_INF = float("-inf")


def _shuffle(v, idx):
    return v.at[idx].get(mode="promise_in_bounds")


def _butterfly(v, iota, op):
    for k in (1, 2, 4, 8):
        v = op(v, _shuffle(v, iota ^ k))
    return v


def _sc_body(x_hbm, out_hbm, buf, res_vmem, sem0, sem1, sem2):
    cid = lax.axis_index("c")
    sid = lax.axis_index("s")
    wid = cid * 16 + sid

    iota = lax.iota(jnp.int32, LANES)
    res = jnp.zeros((LANES,), jnp.float32)

    sems = (sem0, sem1, sem2)
    n_chunks = ROWS_PER_WORKER * CPR
    copies = [None] * NBUF
    row0 = wid * ROWS_PER_WORKER

    def _issue(g):
        slot = g % NBUF
        return pltpu.async_copy(
            x_hbm.at[row0 + g // CPR, pl.ds((g % CPR) * CHUNK_B, CHUNK_B)],
            buf.at[pl.ds(slot * CHUNK_B, CHUNK_B)],
            sems[slot],
        )

    for p in range(NBUF - 1):
        copies[p] = _issue(p)

    ninf = jnp.full((LANES,), _NEG_INF)
    zero = jnp.zeros((LANES,), jnp.float32)

    for j in range(ROWS_PER_WORKER):
        carry = (ninf,) * (2 * K_ACC) + (zero,) * K_ACC
        for c in range(CPR):
            g = j * CPR + c
            if g + NBUF - 1 < n_chunks:
                copies[(g + NBUF - 1) % NBUF] = _issue(g + NBUF - 1)
            copies[g % NBUF].wait()
            slot_base = (g % NBUF) * CHUNK_B

            def fused(i, carry):
                m1s = list(carry[:K_ACC])
                m2s = list(carry[K_ACC:2 * K_ACC])
                accs = list(carry[2 * K_ACC:])
                b = slot_base + i
                for t in range(UNROLL):
                    k = t % K_ACC
                    v = buf[b, t // 8, pl.ds((t % 8) * LANES, LANES)]
                    m2s[k] = jnp.maximum(m2s[k], jnp.minimum(m1s[k], v))
                    m1s[k] = jnp.maximum(m1s[k], v)
                    accs[k] = accs[k] + jnp.exp(v)
                return tuple(m1s) + tuple(m2s) + tuple(accs)

            carry = lax.fori_loop(0, N_ITERS_CHUNK, fused, carry)

        pairs = [(carry[k], carry[K_ACC + k]) for k in range(K_ACC)]
        while len(pairs) > 1:
            nxt_pairs = []
            for p in range(0, len(pairs), 2):
                (a1, a2), (b1, b2) = pairs[p], pairs[p + 1]
                nxt_pairs.append((
                    jnp.maximum(a1, b1),
                    jnp.maximum(jnp.minimum(a1, b1), jnp.maximum(a2, b2)),
                ))
            pairs = nxt_pairs
        m1v, m2v = pairs[0]

        m1b = _butterfly(m1v, iota, jnp.maximum)
        first = _butterfly(
            jnp.where(m1v == m1b, iota, jnp.int32(LANES)), iota, jnp.minimum
        )
        m2b = _butterfly(jnp.where(iota == first, m2v, m1v), iota, jnp.maximum)

        accs = list(carry[2 * K_ACC:])
        while len(accs) > 1:
            accs = [accs[p] + accs[p + 1] for p in range(0, len(accs), 2)]
        sv = _butterfly(accs[0], iota, jnp.add)

        rv = (jnp.exp(m1b + m2b) * jnp.float32(4.0)) / (sv * sv)
        res = jnp.where(iota == j, rv, res)

    res_vmem[...] = res
    pltpu.sync_copy(res_vmem, out_hbm.at[wid])


def _sc_call(x):
    mesh = plsc.VectorSubcoreMesh(core_axis_name="c", subcore_axis_name="s")
    fn = functools.partial(
        pl.kernel,
        mesh=mesh,
        out_type=jax.ShapeDtypeStruct((N_WORKERS, LANES), jnp.float32),
        scratch_types=[
            pltpu.VMEM((NBUF * CHUNK_B, 8, 128), jnp.float32),
            pltpu.VMEM((LANES,), jnp.float32),
            pltpu.SemaphoreType.DMA,
            pltpu.SemaphoreType.DMA,
            pltpu.SemaphoreType.DMA,
        ],
    )(_sc_body)
    return fn(x)


def kernel(inputs):
    x4 = inputs.reshape(ROWS, COLS // BLK, 8, 128)
    out32 = _sc_call(x4)
    return out32[:, :ROWS_PER_WORKER].reshape(ROWS, 1)


# speedup vs baseline: 4.3165x; 1.4657x over previous
import functools

import jax
import jax.numpy as jnp
from jax import lax
from jax.experimental import pallas as pl
from jax.experimental.pallas import tpu as pltpu
from jax.experimental.pallas import tpu_sc as plsc

ROWS = 128
COLS = 32768
LANES = 16
N_WORKERS = 32
ROWS_PER_WORKER = ROWS // N_WORKERS
VREGS_PER_ROW = COLS // LANES
K_ACC = 8
CHUNK = 8192
CPR = COLS // CHUNK
NBUF = 3
UNROLL = 64
N_ITERS_CHUNK = CHUNK // (UNROLL * LANES)

_NEG_INF = float("-inf")


def _shuffle(v, idx):
    return v.at[idx].get(mode="promise_in_bounds")


def _butterfly(v, iota, op):
    for k in (1, 2, 4, 8):
        v = op(v, _shuffle(v, iota ^ k))
    return v


def _sc_body(x_hbm, out_hbm, buf, res_vmem, sem0, sem1, sem2):
    cid = lax.axis_index("c")
    sid = lax.axis_index("s")
    wid = cid * 16 + sid

    iota = lax.iota(jnp.int32, LANES)
    res = jnp.zeros((LANES,), jnp.float32)

    sems = (sem0, sem1, sem2)
    n_chunks = ROWS_PER_WORKER * CPR
    copies = [None] * NBUF
    row0 = wid * ROWS_PER_WORKER

    def _issue(g):
        slot = g % NBUF
        return pltpu.async_copy(
            x_hbm.at[row0 + g // CPR, pl.ds((g % CPR) * CHUNK, CHUNK)],
            buf.at[pl.ds(slot * CHUNK, CHUNK)],
            sems[slot],
        )

    for p in range(NBUF - 1):
        copies[p] = _issue(p)

    ninf = jnp.full((LANES,), _NEG_INF)
    zero = jnp.zeros((LANES,), jnp.float32)

    for j in range(ROWS_PER_WORKER):
        carry = (ninf,) * (2 * K_ACC) + (zero,) * K_ACC
        for c in range(CPR):
            g = j * CPR + c
            if g + NBUF - 1 < n_chunks:
                copies[(g + NBUF - 1) % NBUF] = _issue(g + NBUF - 1)
            copies[g % NBUF].wait()
            slot_base = (g % NBUF) * CHUNK

            def fused(i, carry):
                m1s = list(carry[:K_ACC])
                m2s = list(carry[K_ACC:2 * K_ACC])
                accs = list(carry[2 * K_ACC:])
                base = slot_base + i * (UNROLL * LANES)
                for t in range(UNROLL):
                    k = t % K_ACC
                    v = buf[pl.ds(base + t * LANES, LANES)]
                    m2s[k] = jnp.maximum(m2s[k], jnp.minimum(m1s[k], v))
                    m1s[k] = jnp.maximum(m1s[k], v)
                    accs[k] = accs[k] + jnp.exp(v)
                return tuple(m1s) + tuple(m2s) + tuple(accs)

            carry = lax.fori_loop(0, N_ITERS_CHUNK, fused, carry)

        pairs = [(carry[k], carry[K_ACC + k]) for k in range(K_ACC)]
        while len(pairs) > 1:
            nxt_pairs = []
            for p in range(0, len(pairs), 2):
                (a1, a2), (b1, b2) = pairs[p], pairs[p + 1]
                nxt_pairs.append((
                    jnp.maximum(a1, b1),
                    jnp.maximum(jnp.minimum(a1, b1), jnp.maximum(a2, b2)),
                ))
            pairs = nxt_pairs
        m1v, m2v = pairs[0]

        m1b = _butterfly(m1v, iota, jnp.maximum)
        first = _butterfly(
            jnp.where(m1v == m1b, iota, jnp.int32(LANES)), iota, jnp.minimum
        )
        m2b = _butterfly(jnp.where(iota == first, m2v, m1v), iota, jnp.maximum)

        accs = list(carry[2 * K_ACC:])
        while len(accs) > 1:
            accs = [accs[p] + accs[p + 1] for p in range(0, len(accs), 2)]
        sv = _butterfly(accs[0], iota, jnp.add)

        rv = (jnp.exp(m1b + m2b) * jnp.float32(4.0)) / (sv * sv)
        res = jnp.where(iota == j, rv, res)

    res_vmem[...] = res
    pltpu.sync_copy(res_vmem, out_hbm.at[wid])


def _sc_call(x):
    mesh = plsc.VectorSubcoreMesh(core_axis_name="c", subcore_axis_name="s")
    fn = functools.partial(
        pl.kernel,
        mesh=mesh,
        out_type=jax.ShapeDtypeStruct((N_WORKERS, LANES), jnp.float32),
        scratch_types=[
            pltpu.VMEM((NBUF * CHUNK,), jnp.float32),
            pltpu.VMEM((LANES,), jnp.float32),
            pltpu.SemaphoreType.DMA,
            pltpu.SemaphoreType.DMA,
            pltpu.SemaphoreType.DMA,
        ],
    )(_sc_body)
    return fn(x)


def kernel(inputs):
    out32 = _sc_call(inputs)
    return out32[:, :ROWS_PER_WORKER].reshape(ROWS, 1)


# speedup vs baseline: 4.4924x; 1.0408x over previous
import functools

import jax
import jax.numpy as jnp
from jax import lax
from jax.experimental import pallas as pl
from jax.experimental.pallas import tpu as pltpu
from jax.experimental.pallas import tpu_sc as plsc

ROWS = 128
COLS = 32768
LANES = 16
N_WORKERS = 32
ROWS_PER_WORKER = ROWS // N_WORKERS
VREGS_PER_ROW = COLS // LANES
K_ACC = 8
CHUNK = 8192
CPR = COLS // CHUNK
NBUF = 3
UNROLL = 32
N_ITERS_CHUNK = CHUNK // (UNROLL * LANES)

_NEG_INF = float("-inf")


def _shuffle(v, idx):
    return v.at[idx].get(mode="promise_in_bounds")


def _butterfly(v, iota, op):
    for k in (1, 2, 4, 8):
        v = op(v, _shuffle(v, iota ^ k))
    return v


def _sc_body(x_hbm, out_hbm, buf, res_vmem, sem0, sem1, sem2):
    cid = lax.axis_index("c")
    sid = lax.axis_index("s")
    wid = cid * 16 + sid

    iota = lax.iota(jnp.int32, LANES)
    res = jnp.zeros((LANES,), jnp.float32)

    sems = (sem0, sem1, sem2)
    n_chunks = ROWS_PER_WORKER * CPR
    copies = [None] * NBUF
    row0 = wid * ROWS_PER_WORKER

    def _issue(g):
        slot = g % NBUF
        return pltpu.async_copy(
            x_hbm.at[row0 + g // CPR, pl.ds((g % CPR) * CHUNK, CHUNK)],
            buf.at[pl.ds(slot * CHUNK, CHUNK)],
            sems[slot],
        )

    for p in range(NBUF - 1):
        copies[p] = _issue(p)

    ninf = jnp.full((LANES,), _NEG_INF)
    zero = jnp.zeros((LANES,), jnp.float32)

    for j in range(ROWS_PER_WORKER):
        carry = (ninf,) * (2 * K_ACC) + (zero,) * K_ACC
        for c in range(CPR):
            g = j * CPR + c
            if g + NBUF - 1 < n_chunks:
                copies[(g + NBUF - 1) % NBUF] = _issue(g + NBUF - 1)
            copies[g % NBUF].wait()
            slot_base = (g % NBUF) * CHUNK

            def fused(i, carry):
                m1s = list(carry[:K_ACC])
                m2s = list(carry[K_ACC:2 * K_ACC])
                accs = list(carry[2 * K_ACC:])
                base = slot_base + i * (UNROLL * LANES)
                for t in range(UNROLL):
                    k = t % K_ACC
                    v = buf[pl.ds(base + t * LANES, LANES)]
                    m2s[k] = jnp.maximum(m2s[k], jnp.minimum(m1s[k], v))
                    m1s[k] = jnp.maximum(m1s[k], v)
                    accs[k] = accs[k] + jnp.exp(v)
                return tuple(m1s) + tuple(m2s) + tuple(accs)

            carry = lax.fori_loop(0, N_ITERS_CHUNK, fused, carry)

        pairs = [(carry[k], carry[K_ACC + k]) for k in range(K_ACC)]
        while len(pairs) > 1:
            nxt_pairs = []
            for p in range(0, len(pairs), 2):
                (a1, a2), (b1, b2) = pairs[p], pairs[p + 1]
                nxt_pairs.append((
                    jnp.maximum(a1, b1),
                    jnp.maximum(jnp.minimum(a1, b1), jnp.maximum(a2, b2)),
                ))
            pairs = nxt_pairs
        m1v, m2v = pairs[0]

        m1b = _butterfly(m1v, iota, jnp.maximum)
        first = _butterfly(
            jnp.where(m1v == m1b, iota, jnp.int32(LANES)), iota, jnp.minimum
        )
        m2b = _butterfly(jnp.where(iota == first, m2v, m1v), iota, jnp.maximum)

        accs = list(carry[2 * K_ACC:])
        while len(accs) > 1:
            accs = [accs[p] + accs[p + 1] for p in range(0, len(accs), 2)]
        sv = _butterfly(accs[0], iota, jnp.add)

        rv = (jnp.exp(m1b + m2b) * jnp.float32(4.0)) / (sv * sv)
        res = jnp.where(iota == j, rv, res)

    res_vmem[...] = res
    pltpu.sync_copy(res_vmem, out_hbm.at[wid])


def _sc_call(x):
    mesh = plsc.VectorSubcoreMesh(core_axis_name="c", subcore_axis_name="s")
    fn = functools.partial(
        pl.kernel,
        mesh=mesh,
        out_type=jax.ShapeDtypeStruct((N_WORKERS, LANES), jnp.float32),
        scratch_types=[
            pltpu.VMEM((NBUF * CHUNK,), jnp.float32),
            pltpu.VMEM((LANES,), jnp.float32),
            pltpu.SemaphoreType.DMA,
            pltpu.SemaphoreType.DMA,
            pltpu.SemaphoreType.DMA,
        ],
    )(_sc_body)
    return fn(x)


def kernel(inputs):
    out32 = _sc_call(inputs)
    return out32[:, :ROWS_PER_WORKER].reshape(ROWS, 1)


# speedup vs baseline: 4.7474x; 1.0568x over previous
import functools

import jax
import jax.numpy as jnp
from jax import lax
from jax.experimental import pallas as pl
from jax.experimental.pallas import tpu as pltpu
from jax.experimental.pallas import tpu_sc as plsc

ROWS = 128
COLS = 32768
LANES = 16
N_WORKERS = 32
ROWS_PER_WORKER = ROWS // N_WORKERS
VREGS_PER_ROW = COLS // LANES
K_ACC = 8
CHUNK = 8192
CPR = COLS // CHUNK
NBUF = 3
UNROLL = 16
N_ITERS_CHUNK = CHUNK // (UNROLL * LANES)

_NEG_INF = float("-inf")


def _shuffle(v, idx):
    return v.at[idx].get(mode="promise_in_bounds")


def _butterfly(v, iota, op):
    for k in (1, 2, 4, 8):
        v = op(v, _shuffle(v, iota ^ k))
    return v


def _sc_body(x_hbm, out_hbm, buf, res_vmem, sem0, sem1, sem2):
    cid = lax.axis_index("c")
    sid = lax.axis_index("s")
    wid = cid * 16 + sid

    iota = lax.iota(jnp.int32, LANES)
    res = jnp.zeros((LANES,), jnp.float32)

    sems = (sem0, sem1, sem2)
    n_chunks = ROWS_PER_WORKER * CPR
    copies = [None] * NBUF
    row0 = wid * ROWS_PER_WORKER

    def _issue(g):
        slot = g % NBUF
        return pltpu.async_copy(
            x_hbm.at[row0 + g // CPR, pl.ds((g % CPR) * CHUNK, CHUNK)],
            buf.at[pl.ds(slot * CHUNK, CHUNK)],
            sems[slot],
        )

    for p in range(NBUF - 1):
        copies[p] = _issue(p)

    ninf = jnp.full((LANES,), _NEG_INF)
    zero = jnp.zeros((LANES,), jnp.float32)

    for j in range(ROWS_PER_WORKER):
        carry = (ninf,) * (2 * K_ACC) + (zero,) * K_ACC
        for c in range(CPR):
            g = j * CPR + c
            if g + NBUF - 1 < n_chunks:
                copies[(g + NBUF - 1) % NBUF] = _issue(g + NBUF - 1)
            copies[g % NBUF].wait()
            slot_base = (g % NBUF) * CHUNK

            def fused(i, carry):
                m1s = list(carry[:K_ACC])
                m2s = list(carry[K_ACC:2 * K_ACC])
                accs = list(carry[2 * K_ACC:])
                base = pl.multiple_of(
                    slot_base + i * (UNROLL * LANES), UNROLL * LANES
                )
                for t in range(UNROLL):
                    k = t % K_ACC
                    v = buf[pl.ds(base + t * LANES, LANES)]
                    m2s[k] = jnp.maximum(m2s[k], jnp.minimum(m1s[k], v))
                    m1s[k] = jnp.maximum(m1s[k], v)
                    accs[k] = accs[k] + jnp.exp(v)
                return tuple(m1s) + tuple(m2s) + tuple(accs)

            carry = lax.fori_loop(0, N_ITERS_CHUNK, fused, carry)

        pairs = [(carry[k], carry[K_ACC + k]) for k in range(K_ACC)]
        while len(pairs) > 1:
            nxt_pairs = []
            for p in range(0, len(pairs), 2):
                (a1, a2), (b1, b2) = pairs[p], pairs[p + 1]
                nxt_pairs.append((
                    jnp.maximum(a1, b1),
                    jnp.maximum(jnp.minimum(a1, b1), jnp.maximum(a2, b2)),
                ))
            pairs = nxt_pairs
        m1v, m2v = pairs[0]

        m1b = _butterfly(m1v, iota, jnp.maximum)
        first = _butterfly(
            jnp.where(m1v == m1b, iota, jnp.int32(LANES)), iota, jnp.minimum
        )
        m2b = _butterfly(jnp.where(iota == first, m2v, m1v), iota, jnp.maximum)

        accs = list(carry[2 * K_ACC:])
        while len(accs) > 1:
            accs = [accs[p] + accs[p + 1] for p in range(0, len(accs), 2)]
        sv = _butterfly(accs[0], iota, jnp.add)

        rv = (jnp.exp(m1b + m2b) * jnp.float32(4.0)) / (sv * sv)
        res = jnp.where(iota == j, rv, res)

    res_vmem[...] = res
    pltpu.sync_copy(res_vmem, out_hbm.at[wid])


def _sc_call(x):
    mesh = plsc.VectorSubcoreMesh(core_axis_name="c", subcore_axis_name="s")
    fn = functools.partial(
        pl.kernel,
        mesh=mesh,
        out_type=jax.ShapeDtypeStruct((N_WORKERS, LANES), jnp.float32),
        scratch_types=[
            pltpu.VMEM((NBUF * CHUNK,), jnp.float32),
            pltpu.VMEM((LANES,), jnp.float32),
            pltpu.SemaphoreType.DMA,
            pltpu.SemaphoreType.DMA,
            pltpu.SemaphoreType.DMA,
        ],
    )(_sc_body)
    return fn(x)


def kernel(inputs):
    out32 = _sc_call(inputs)
    return out32[:, :ROWS_PER_WORKER].reshape(ROWS, 1)


# speedup vs baseline: 4.7478x; 1.0001x over previous
import functools

import jax
import jax.numpy as jnp
from jax import lax
from jax.experimental import pallas as pl
from jax.experimental.pallas import tpu as pltpu
from jax.experimental.pallas import tpu_sc as plsc

ROWS = 128
COLS = 32768
LANES = 16
N_WORKERS = 32
ROWS_PER_WORKER = ROWS // N_WORKERS
VREGS_PER_ROW = COLS // LANES
K_ACC = 4
CHUNK = 8192
CPR = COLS // CHUNK
NBUF = 3
UNROLL = 16
N_ITERS_CHUNK = CHUNK // (UNROLL * LANES)

_NEG_INF = float("-inf")


def _shuffle(v, idx):
    return v.at[idx].get(mode="promise_in_bounds")


def _butterfly(v, iota, op):
    for k in (1, 2, 4, 8):
        v = op(v, _shuffle(v, iota ^ k))
    return v


def _sc_body(x_hbm, out_hbm, buf, res_vmem, sem0, sem1, sem2):
    cid = lax.axis_index("c")
    sid = lax.axis_index("s")
    wid = cid * 16 + sid

    iota = lax.iota(jnp.int32, LANES)
    res = jnp.zeros((LANES,), jnp.float32)

    sems = (sem0, sem1, sem2)
    n_chunks = ROWS_PER_WORKER * CPR
    copies = [None] * NBUF
    row0 = wid * ROWS_PER_WORKER

    def _issue(g):
        slot = g % NBUF
        return pltpu.async_copy(
            x_hbm.at[row0 + g // CPR, pl.ds((g % CPR) * CHUNK, CHUNK)],
            buf.at[pl.ds(slot * CHUNK, CHUNK)],
            sems[slot],
        )

    for p in range(NBUF - 1):
        copies[p] = _issue(p)

    ninf = jnp.full((LANES,), _NEG_INF)
    zero = jnp.zeros((LANES,), jnp.float32)

    for j in range(ROWS_PER_WORKER):
        carry = (ninf,) * (2 * K_ACC) + (zero,) * K_ACC
        for c in range(CPR):
            g = j * CPR + c
            if g + NBUF - 1 < n_chunks:
                copies[(g + NBUF - 1) % NBUF] = _issue(g + NBUF - 1)
            copies[g % NBUF].wait()
            slot_base = (g % NBUF) * CHUNK

            def fused(i, carry):
                m1s = list(carry[:K_ACC])
                m2s = list(carry[K_ACC:2 * K_ACC])
                accs = list(carry[2 * K_ACC:])
                base = pl.multiple_of(
                    slot_base + i * (UNROLL * LANES), UNROLL * LANES
                )
                for t in range(UNROLL):
                    k = t % K_ACC
                    v = buf[pl.ds(base + t * LANES, LANES)]
                    m2s[k] = jnp.maximum(m2s[k], jnp.minimum(m1s[k], v))
                    m1s[k] = jnp.maximum(m1s[k], v)
                    accs[k] = accs[k] + jnp.exp(v)
                return tuple(m1s) + tuple(m2s) + tuple(accs)

            carry = lax.fori_loop(0, N_ITERS_CHUNK, fused, carry)

        pairs = [(carry[k], carry[K_ACC + k]) for k in range(K_ACC)]
        while len(pairs) > 1:
            nxt_pairs = []
            for p in range(0, len(pairs), 2):
                (a1, a2), (b1, b2) = pairs[p], pairs[p + 1]
                nxt_pairs.append((
                    jnp.maximum(a1, b1),
                    jnp.maximum(jnp.minimum(a1, b1), jnp.maximum(a2, b2)),
                ))
            pairs = nxt_pairs
        m1v, m2v = pairs[0]

        m1b = _butterfly(m1v, iota, jnp.maximum)
        first = _butterfly(
            jnp.where(m1v == m1b, iota, jnp.int32(LANES)), iota, jnp.minimum
        )
        m2b = _butterfly(jnp.where(iota == first, m2v, m1v), iota, jnp.maximum)

        accs = list(carry[2 * K_ACC:])
        while len(accs) > 1:
            accs = [accs[p] + accs[p + 1] for p in range(0, len(accs), 2)]
        sv = _butterfly(accs[0], iota, jnp.add)

        rv = (jnp.exp(m1b + m2b) * jnp.float32(4.0)) / (sv * sv)
        res = jnp.where(iota == j, rv, res)

    res_vmem[...] = res
    pltpu.sync_copy(res_vmem, out_hbm.at[wid])


def _sc_call(x):
    mesh = plsc.VectorSubcoreMesh(core_axis_name="c", subcore_axis_name="s")
    fn = functools.partial(
        pl.kernel,
        mesh=mesh,
        out_type=jax.ShapeDtypeStruct((N_WORKERS, LANES), jnp.float32),
        scratch_types=[
            pltpu.VMEM((NBUF * CHUNK,), jnp.float32),
            pltpu.VMEM((LANES,), jnp.float32),
            pltpu.SemaphoreType.DMA,
            pltpu.SemaphoreType.DMA,
            pltpu.SemaphoreType.DMA,
        ],
    )(_sc_body)
    return fn(x)


def kernel(inputs):
    out32 = _sc_call(inputs)
    return out32[:, :ROWS_PER_WORKER].reshape(ROWS, 1)
